# baseline (speedup 1.0000x reference)
import functools

import jax
import jax.numpy as jnp
from jax import lax
from jax.experimental import pallas as pl
from jax.experimental.pallas import tpu as pltpu
from jax.experimental.pallas import tpu_sc as plsc

N = 100000
E = 1600000
DIM = 32
H = 16
NC = 2
NS = 16
CH = 128
NCH = 784
TPW = NCH * CH
E_PAD = NS * TPW
N_ACC = 100352
ZPT = N_ACC // NS
NB = 4
Q = 2 * NB


def _mp_body(edge_split, xt, ept, srcoff, dstp, zrows, out, sd, xg, ev, ms,
             acc, sem_sd, sem_g, sem_e, sem_sc):
    c = lax.axis_index("c")
    s = lax.axis_index("s")
    pltpu.sync_copy(zrows, acc.at[pl.ds(s * ZPT, ZPT)])
    plsc.subcore_barrier()
    if edge_split:
        tbase = (c * NS + s) * (TPW // NC)
        nch = NCH // NC
    else:
        tbase = s * TPW
        nch = NCH

    def sd_copies(j, q):
        ebase = tbase + j * CH
        if edge_split:
            src_sl = srcoff.at[pl.ds(ebase, CH)]
        else:
            src_sl = srcoff.at[c, pl.ds(ebase, CH)]
        return (pltpu.make_async_copy(src_sl, sd.at[q, 0], sem_sd.at[q]),
                pltpu.make_async_copy(dstp.at[pl.ds(ebase, CH)], sd.at[q, 1],
                                      sem_sd.at[q]))

    def ev_copy(j, b):
        rbase = (tbase + j * CH) // 8
        if edge_split:
            src_sl = ept.at[pl.ds(rbase, CH // 8)]
        else:
            src_sl = ept.at[c, pl.ds(rbase, CH // 8)]
        return pltpu.make_async_copy(src_sl, ev.at[b], sem_e.at[b])

    def gather_copy(q, b):
        return pltpu.make_async_copy(xt.at[sd.at[q, 0]], xg.at[b],
                                     sem_g.at[b])

    def scatter_copy(q, b):
        return pltpu.make_async_copy(ms.at[b], acc.at[sd.at[q, 1]],
                                     sem_sc.at[b])

    for k in range(NB):
        for d in sd_copies(k, k):
            d.start()
        ev_copy(k, k).start()
    for d in sd_copies(0, 0):
        d.wait()
    gather_copy(0, 0).start()

    @pl.loop(0, nch, step=Q)
    def _outer(j0):
        for b2 in range(Q):
            j = j0 + b2
            b = b2 % NB
            q = b2

            @pl.when(j >= NB)
            def _():
                scatter_copy((b2 - NB) % Q, b).wait()

            gather_copy(q, b).wait()
            ev_copy(j, b).wait()

            @plsc.parallel_loop(0, CH, 1, unroll=8)
            def _row(i):
                ef = ev[b, i // 8, pl.ds((i % 8) * H, H)]
                ms[b, i, :] = jnp.maximum(xg[b, i, :] + ef, 0.0)

            scatter_copy(q, b).start(add=True)

            @pl.when(j + NB < nch)
            def _():
                for d in sd_copies(j + NB, (b2 + NB) % Q):
                    d.start()
                ev_copy(j + NB, b).start()

            @pl.when(j + 1 < nch)
            def _():
                for d in sd_copies(j + 1, (b2 + 1) % Q):
                    d.wait()
                gather_copy((b2 + 1) % Q, (b2 + 1) % NB).start()

    for b2 in range(NB, Q):
        scatter_copy(b2, b2 - NB).wait()

    plsc.subcore_barrier()
    pltpu.sync_copy(acc.at[pl.ds(s * ZPT, ZPT)], out.at[c, pl.ds(s * ZPT, ZPT)])


def _mp_call(edge_split, xt, ept, srcoff, dstp, zrows):
    mesh = plsc.VectorSubcoreMesh(
        core_axis_name="c", subcore_axis_name="s", num_cores=NC, num_subcores=NS)
    f = functools.partial(
        pl.kernel,
        out_type=jax.ShapeDtypeStruct((NC, N_ACC, H), jnp.float32),
        mesh=mesh,
        scratch_types=[
            pltpu.VMEM((Q, 2, CH), jnp.int32),
            pltpu.VMEM((NB, CH, H), jnp.float32),
            pltpu.VMEM((NB, CH // 8, 8 * H), jnp.float32),
            pltpu.VMEM((NB, CH, H), jnp.float32),
            pltpu.VMEM_SHARED((N_ACC, H), jnp.float32),
            pltpu.SemaphoreType.DMA((Q,)),
            pltpu.SemaphoreType.DMA((NB,)),
            pltpu.SemaphoreType.DMA((NB,)),
            pltpu.SemaphoreType.DMA((NB,)),
        ],
        compiler_params=pltpu.CompilerParams(use_tc_tiling_on_sc=False),
        name="gin_message_passing",
    )(functools.partial(_mp_body, edge_split))
    return f(xt, ept, srcoff, dstp, zrows)


_mp = jax.jit(functools.partial(_mp_call, False))
_mp_es = jax.jit(functools.partial(_mp_call, True))

BR = 4096


def _dot(a, b):
    return jnp.dot(a, b, precision=lax.Precision.HIGHEST)


def _emlp_body2(a_ref, w1_ref, b1_ref, w2_ref, b2_ref, o_ref):
    e1 = jnp.maximum(_dot(a_ref[...], w1_ref[...]) + b1_ref[...], 0.0)
    o_ref[0] = _dot(e1, w2_ref[0]) + b2_ref[0]
    o_ref[1] = _dot(e1, w2_ref[1]) + b2_ref[1]


def _emlp_body1(a_ref, w1_ref, b1_ref, w2_ref, b2_ref, o_ref):
    e1 = jnp.maximum(_dot(a_ref[...], w1_ref[...]) + b1_ref[...], 0.0)
    o_ref[...] = _dot(e1, w2_ref[...]) + b2_ref[...]


def _edge_mlp(nout, eaP, w1b, b1b, w2b, b2b):
    R = E_PAD // 8
    if nout == 2:
        out_shape = jax.ShapeDtypeStruct((NC, R, 8 * H), jnp.float32)
        o_spec = pl.BlockSpec((NC, BR, 8 * H), lambda i: (0, i, 0))
        w2_spec = pl.BlockSpec(w2b.shape, lambda i: (0, 0, 0))
        b2_spec = pl.BlockSpec(b2b.shape, lambda i: (0, 0, 0))
        body = _emlp_body2
    else:
        out_shape = jax.ShapeDtypeStruct((R, 8 * H), jnp.float32)
        o_spec = pl.BlockSpec((BR, 8 * H), lambda i: (i, 0))
        w2_spec = pl.BlockSpec(w2b.shape, lambda i: (0, 0))
        b2_spec = pl.BlockSpec(b2b.shape, lambda i: (0, 0))
        body = _emlp_body1
    return pl.pallas_call(
        body,
        grid=(R // BR,),
        in_specs=[
            pl.BlockSpec((BR, 32), lambda i: (i, 0)),
            pl.BlockSpec(w1b.shape, lambda i: (0, 0)),
            pl.BlockSpec(b1b.shape, lambda i: (0, 0)),
            w2_spec,
            b2_spec,
        ],
        out_specs=o_spec,
        out_shape=out_shape,
        name="edge_mlp",
    )(eaP, w1b, b1b, w2b, b2b)


def _blockdiag8(m):
    a, b = m.shape
    out = jnp.zeros((8, a, 8, b), m.dtype)
    out = out.at[jnp.arange(8), :, jnp.arange(8), :].set(m)
    return out.reshape(8 * a, 8 * b)


def _set2set(x, batch, p, num_graphs, steps=6):
    Hd = x.shape[1]
    ohb = batch[:, None] == jnp.arange(num_graphs, dtype=batch.dtype)[None, :]
    oh = ohb.astype(x.dtype)
    q_star = jnp.zeros((num_graphs, 2 * Hd), dtype=x.dtype)
    h = jnp.zeros((num_graphs, Hd), dtype=x.dtype)
    c = jnp.zeros((num_graphs, Hd), dtype=x.dtype)
    for _ in range(steps):
        gates = q_star @ p['w_ih'].T + p['b_ih'] + h @ p['w_hh'].T + p['b_hh']
        i, f, g, o = jnp.split(gates, 4, axis=1)
        i = jax.nn.sigmoid(i); f = jax.nn.sigmoid(f)
        g = jnp.tanh(g); o = jax.nn.sigmoid(o)
        c = f * c + i * g
        h = o * jnp.tanh(c)
        q = h
        e = jnp.sum(x * (oh @ q), axis=-1)
        emax = jnp.max(jnp.where(ohb, e[:, None], -1e30), axis=0)
        ee = jnp.exp(e - oh @ emax)
        denom = ee @ oh
        a = ee / (oh @ denom + 1e-16)
        r = jnp.einsum('ng,nd->gd', oh, a[:, None] * x)
        q_star = jnp.concatenate([q, r], axis=1)
    return q_star


def kernel(x, edge_index, edge_attr, batch, params):
    f32 = jnp.float32
    src = edge_index[0]
    dst = edge_index[1]
    npad = E_PAD - E
    srcp = jnp.concatenate([src, (jnp.arange(npad, dtype=jnp.int32) * 997) % N])
    dstp = jnp.concatenate([dst, N + (jnp.arange(npad, dtype=jnp.int32) % 8)])
    srcoff = srcp[None, :] + (jnp.arange(NC, dtype=jnp.int32) * N)[:, None]
    zrows = jnp.zeros((ZPT, H), f32)
    eaP = jnp.pad(edge_attr, ((0, npad), (0, 0))).reshape(E_PAD // 8, 32)

    h = x
    for p in params['convs']:
        d1 = p['mlp1']['w'].shape[0]
        w1t = jnp.pad(p['be1']['w'].T, ((0, 0), (0, DIM - d1)))
        w1b = _blockdiag8(w1t)
        b1b = jnp.tile(jnp.pad(p['be1']['b'], (0, DIM - d1)), 8)[None, :]
        w2t = p['be2']['w'].T
        if d1 == DIM:
            w2b = jnp.stack([_blockdiag8(w2t[:, :H]),
                             _blockdiag8(w2t[:, H:])])
            b2b = jnp.stack([jnp.tile(p['be2']['b'][:H], 8),
                             jnp.tile(p['be2']['b'][H:], 8)])[:, None, :]
            ept = _edge_mlp(2, eaP, w1b, b1b, w2b, b2b)
            xt = h.reshape(N, NC, H).transpose(1, 0, 2).reshape(NC * N, H)
            agg2 = _mp(xt, ept, srcoff, dstp, zrows)
            agg = jnp.concatenate([agg2[0, :N], agg2[1, :N]], axis=1)
        else:
            w2tp = jnp.pad(w2t, ((0, DIM - d1), (0, H - d1)))
            w2b = _blockdiag8(w2tp)
            b2b = jnp.tile(jnp.pad(p['be2']['b'], (0, H - d1)), 8)[None, :]
            ept = _edge_mlp(1, eaP, w1b, b1b, w2b, b2b)
            xt = jnp.pad(h, ((0, 0), (0, H - d1)))
            agg2 = _mp_es(xt, ept, srcp, dstp, zrows)
            agg = (agg2[0, :N, :d1] + agg2[1, :N, :d1])
        hh = (1.0 + p['eps']) * h + agg
        hh = jax.nn.relu(hh @ p['mlp1']['w'].T + p['mlp1']['b'])
        hh = hh @ p['mlp2']['w'].T + p['mlp2']['b']
        h = jax.nn.relu(hh)

    q = _set2set(h, batch, params['s2s'], 64, steps=6)
    out = jax.nn.relu(q @ params['fc1']['w'].T + params['fc1']['b'])
    out = out @ params['fc4']['w'].T + params['fc4']['b']
    return out


# speedup vs baseline: 1.1164x; 1.1164x over previous
import functools

import jax
import jax.numpy as jnp
from jax import lax
from jax.experimental import pallas as pl
from jax.experimental.pallas import tpu as pltpu
from jax.experimental.pallas import tpu_sc as plsc

N = 100000
E = 1600000
DIM = 32
H = 16
NC = 2
NS = 16
CH = 128
NCH = 784
TPW = NCH * CH
E_PAD = NS * TPW
N_ACC = 100352
ZPT = N_ACC // NS
NB = 4
Q = 2 * NB


def _mp_body(edge_split, xt, ept, srcoff, dstp, zrows, out, sd, xg, ev, ms,
             acc, sem_sd, sem_g, sem_e, sem_sc):
    c = lax.axis_index("c")
    s = lax.axis_index("s")
    pltpu.sync_copy(zrows, acc.at[pl.ds(s * ZPT, ZPT)])
    plsc.subcore_barrier()
    if edge_split:
        tbase = (c * NS + s) * (TPW // NC)
        nch = NCH // NC
    else:
        tbase = s * TPW
        nch = NCH

    def sd_copies(j, q):
        ebase = tbase + j * CH
        if edge_split:
            src_sl = srcoff.at[pl.ds(ebase, CH)]
        else:
            src_sl = srcoff.at[c, pl.ds(ebase, CH)]
        return (pltpu.make_async_copy(src_sl, sd.at[q, 0], sem_sd.at[q]),
                pltpu.make_async_copy(dstp.at[pl.ds(ebase, CH)], sd.at[q, 1],
                                      sem_sd.at[q]))

    def ev_copy(j, b):
        rbase = (tbase + j * CH) // 8
        if edge_split:
            src_sl = ept.at[pl.ds(rbase, CH // 8)]
        else:
            src_sl = ept.at[c, pl.ds(rbase, CH // 8)]
        return pltpu.make_async_copy(src_sl, ev.at[b], sem_e.at[b])

    def gather_copy(q, b):
        return pltpu.make_async_copy(xt.at[sd.at[q, 0]], xg.at[b],
                                     sem_g.at[b])

    def scatter_copy(q, b):
        return pltpu.make_async_copy(ms.at[b], acc.at[sd.at[q, 1]],
                                     sem_sc.at[b])

    for k in range(NB):
        for d in sd_copies(k, k):
            d.start()
        ev_copy(k, k).start()
    for d in sd_copies(0, 0):
        d.wait()
    gather_copy(0, 0).start()

    @pl.loop(0, nch, step=Q)
    def _outer(j0):
        for b2 in range(Q):
            j = j0 + b2
            b = b2 % NB
            q = b2

            @pl.when(j >= NB)
            def _():
                scatter_copy((b2 - NB) % Q, b).wait()

            gather_copy(q, b).wait()
            ev_copy(j, b).wait()

            @plsc.parallel_loop(0, CH, 1, unroll=8)
            def _row(i):
                ef = ev[b, i // 8, pl.ds((i % 8) * H, H)]
                ms[b, i, :] = jnp.maximum(xg[b, i, :] + ef, 0.0)

            scatter_copy(q, b).start(add=True)

            @pl.when(j + NB < nch)
            def _():
                for d in sd_copies(j + NB, (b2 + NB) % Q):
                    d.start()
                ev_copy(j + NB, b).start()

            @pl.when(j + 1 < nch)
            def _():
                for d in sd_copies(j + 1, (b2 + 1) % Q):
                    d.wait()
                gather_copy((b2 + 1) % Q, (b2 + 1) % NB).start()

    for b2 in range(NB, Q):
        scatter_copy(b2, b2 - NB).wait()

    plsc.subcore_barrier()
    pltpu.sync_copy(acc.at[pl.ds(s * ZPT, ZPT)], out.at[c, pl.ds(s * ZPT, ZPT)])


def _mp_call(edge_split, xt, ept, srcoff, dstp, zrows):
    mesh = plsc.VectorSubcoreMesh(
        core_axis_name="c", subcore_axis_name="s", num_cores=NC, num_subcores=NS)
    f = functools.partial(
        pl.kernel,
        out_type=jax.ShapeDtypeStruct((NC, N_ACC, H), jnp.float32),
        mesh=mesh,
        scratch_types=[
            pltpu.VMEM((Q, 2, CH), jnp.int32),
            pltpu.VMEM((NB, CH, H), jnp.float32),
            pltpu.VMEM((NB, CH // 8, 8 * H), jnp.float32),
            pltpu.VMEM((NB, CH, H), jnp.float32),
            pltpu.VMEM_SHARED((N_ACC, H), jnp.float32),
            pltpu.SemaphoreType.DMA((Q,)),
            pltpu.SemaphoreType.DMA((NB,)),
            pltpu.SemaphoreType.DMA((NB,)),
            pltpu.SemaphoreType.DMA((NB,)),
        ],
        compiler_params=pltpu.CompilerParams(use_tc_tiling_on_sc=False),
        name="gin_message_passing",
    )(functools.partial(_mp_body, edge_split))
    return f(xt, ept, srcoff, dstp, zrows)


_mp = jax.jit(functools.partial(_mp_call, False))
_mp_es = jax.jit(functools.partial(_mp_call, True))

BR = 4096


def _dot(a, b):
    bf = jnp.bfloat16
    a_hi = a.astype(bf)
    b_hi = b.astype(bf)
    a_lo = (a - a_hi.astype(jnp.float32)).astype(bf)
    b_lo = (b - b_hi.astype(jnp.float32)).astype(bf)
    f32 = jnp.float32
    return (jnp.dot(a_hi, b_hi, preferred_element_type=f32)
            + jnp.dot(a_hi, b_lo, preferred_element_type=f32)
            + jnp.dot(a_lo, b_hi, preferred_element_type=f32))


def _emlp_body2(a_ref, w1_ref, b1_ref, w2_ref, b2_ref, o_ref):
    e1 = jnp.maximum(_dot(a_ref[...], w1_ref[...]) + b1_ref[...], 0.0)
    o_ref[0] = _dot(e1, w2_ref[0]) + b2_ref[0]
    o_ref[1] = _dot(e1, w2_ref[1]) + b2_ref[1]


def _emlp_body1(a_ref, w1_ref, b1_ref, w2_ref, b2_ref, o_ref):
    e1 = jnp.maximum(_dot(a_ref[...], w1_ref[...]) + b1_ref[...], 0.0)
    o_ref[...] = _dot(e1, w2_ref[...]) + b2_ref[...]


def _edge_mlp(nout, eaP, w1b, b1b, w2b, b2b):
    R = E_PAD // 8
    if nout == 2:
        out_shape = jax.ShapeDtypeStruct((NC, R, 8 * H), jnp.float32)
        o_spec = pl.BlockSpec((NC, BR, 8 * H), lambda i: (0, i, 0))
        w2_spec = pl.BlockSpec(w2b.shape, lambda i: (0, 0, 0))
        b2_spec = pl.BlockSpec(b2b.shape, lambda i: (0, 0, 0))
        body = _emlp_body2
    else:
        out_shape = jax.ShapeDtypeStruct((R, 8 * H), jnp.float32)
        o_spec = pl.BlockSpec((BR, 8 * H), lambda i: (i, 0))
        w2_spec = pl.BlockSpec(w2b.shape, lambda i: (0, 0))
        b2_spec = pl.BlockSpec(b2b.shape, lambda i: (0, 0))
        body = _emlp_body1
    return pl.pallas_call(
        body,
        grid=(R // BR,),
        in_specs=[
            pl.BlockSpec((BR, 32), lambda i: (i, 0)),
            pl.BlockSpec(w1b.shape, lambda i: (0, 0)),
            pl.BlockSpec(b1b.shape, lambda i: (0, 0)),
            w2_spec,
            b2_spec,
        ],
        out_specs=o_spec,
        out_shape=out_shape,
        name="edge_mlp",
    )(eaP, w1b, b1b, w2b, b2b)


def _blockdiag8(m):
    a, b = m.shape
    out = jnp.zeros((8, a, 8, b), m.dtype)
    out = out.at[jnp.arange(8), :, jnp.arange(8), :].set(m)
    return out.reshape(8 * a, 8 * b)


def _set2set(x, batch, p, num_graphs, steps=6):
    Hd = x.shape[1]
    ohb = batch[:, None] == jnp.arange(num_graphs, dtype=batch.dtype)[None, :]
    oh = ohb.astype(x.dtype)
    q_star = jnp.zeros((num_graphs, 2 * Hd), dtype=x.dtype)
    h = jnp.zeros((num_graphs, Hd), dtype=x.dtype)
    c = jnp.zeros((num_graphs, Hd), dtype=x.dtype)
    for _ in range(steps):
        gates = q_star @ p['w_ih'].T + p['b_ih'] + h @ p['w_hh'].T + p['b_hh']
        i, f, g, o = jnp.split(gates, 4, axis=1)
        i = jax.nn.sigmoid(i); f = jax.nn.sigmoid(f)
        g = jnp.tanh(g); o = jax.nn.sigmoid(o)
        c = f * c + i * g
        h = o * jnp.tanh(c)
        q = h
        e = jnp.sum(x * (oh @ q), axis=-1)
        emax = jnp.max(jnp.where(ohb, e[:, None], -1e30), axis=0)
        ee = jnp.exp(e - oh @ emax)
        denom = ee @ oh
        a = ee / (oh @ denom + 1e-16)
        r = jnp.einsum('ng,nd->gd', oh, a[:, None] * x)
        q_star = jnp.concatenate([q, r], axis=1)
    return q_star


def kernel(x, edge_index, edge_attr, batch, params):
    f32 = jnp.float32
    src = edge_index[0]
    dst = edge_index[1]
    npad = E_PAD - E
    srcp = jnp.concatenate([src, (jnp.arange(npad, dtype=jnp.int32) * 997) % N])
    dstp = jnp.concatenate([dst, N + (jnp.arange(npad, dtype=jnp.int32) % 8)])
    srcoff = srcp[None, :] + (jnp.arange(NC, dtype=jnp.int32) * N)[:, None]
    zrows = jnp.zeros((ZPT, H), f32)
    eaP = jnp.pad(edge_attr, ((0, npad), (0, 0))).reshape(E_PAD // 8, 32)

    h = x
    for p in params['convs']:
        d1 = p['mlp1']['w'].shape[0]
        w1t = jnp.pad(p['be1']['w'].T, ((0, 0), (0, DIM - d1)))
        w1b = _blockdiag8(w1t)
        b1b = jnp.tile(jnp.pad(p['be1']['b'], (0, DIM - d1)), 8)[None, :]
        w2t = p['be2']['w'].T
        if d1 == DIM:
            w2b = jnp.stack([_blockdiag8(w2t[:, :H]),
                             _blockdiag8(w2t[:, H:])])
            b2b = jnp.stack([jnp.tile(p['be2']['b'][:H], 8),
                             jnp.tile(p['be2']['b'][H:], 8)])[:, None, :]
            ept = _edge_mlp(2, eaP, w1b, b1b, w2b, b2b)
            xt = h.reshape(N, NC, H).transpose(1, 0, 2).reshape(NC * N, H)
            agg2 = _mp(xt, ept, srcoff, dstp, zrows)
            agg = jnp.concatenate([agg2[0, :N], agg2[1, :N]], axis=1)
        else:
            w2tp = jnp.pad(w2t, ((0, DIM - d1), (0, H - d1)))
            w2b = _blockdiag8(w2tp)
            b2b = jnp.tile(jnp.pad(p['be2']['b'], (0, H - d1)), 8)[None, :]
            ept = _edge_mlp(1, eaP, w1b, b1b, w2b, b2b)
            xt = jnp.pad(h, ((0, 0), (0, H - d1)))
            agg2 = _mp_es(xt, ept, srcp, dstp, zrows)
            agg = (agg2[0, :N, :d1] + agg2[1, :N, :d1])
        hh = (1.0 + p['eps']) * h + agg
        hh = jax.nn.relu(hh @ p['mlp1']['w'].T + p['mlp1']['b'])
        hh = hh @ p['mlp2']['w'].T + p['mlp2']['b']
        h = jax.nn.relu(hh)

    q = _set2set(h, batch, params['s2s'], 64, steps=6)
    out = jax.nn.relu(q @ params['fc1']['w'].T + params['fc1']['b'])
    out = out @ params['fc4']['w'].T + params['fc4']['b']
    return out
